# baseline (speedup 1.0000x reference)
import jax
import jax.numpy as jnp
from jax.experimental import pallas as pl
from jax.experimental.pallas import tpu as pltpu

NUM_CLASSES = 1000
CPAD = 1024
HIDDEN_DIM = 768
SEQ_LEN = 77
BATCH = 1024
BB = 256


def _body(sp_ref, whi_ref, wlo_ref, g_ref, b_ref, o_ref):
    i = pl.program_id(0)
    sp = sp_ref[...]
    cols = jax.lax.broadcasted_iota(jnp.int32, (BB, CPAD), 1)
    oh = jnp.where(cols == sp, 1.0, 0.0).astype(jnp.bfloat16)
    rows = jnp.dot(
        oh, whi_ref[...], preferred_element_type=jnp.float32
    ) + jnp.dot(oh, wlo_ref[...], preferred_element_type=jnp.float32) * (1.0 / 512.0)
    mu = jnp.mean(rows, axis=-1, keepdims=True)
    var = jnp.mean(jnp.square(rows - mu), axis=-1, keepdims=True)
    o_ref[pl.ds(i * BB, BB), :] = (
        (rows - mu) * jax.lax.rsqrt(var + 1e-5) * g_ref[...] + b_ref[...]
    )


def kernel(species, W, gamma, beta):
    species2 = species.astype(jnp.int32).reshape(BATCH, 1)
    Wp = jnp.concatenate(
        [W, jnp.zeros((CPAD - NUM_CLASSES, HIDDEN_DIM), jnp.float32)], axis=0
    )
    w_hi = Wp.astype(jnp.bfloat16)
    w_lo = ((Wp - w_hi.astype(jnp.float32)) * 512.0).astype(jnp.bfloat16)
    emb = pl.pallas_call(
        _body,
        grid=(BATCH // BB,),
        in_specs=[
            pl.BlockSpec((BB, 1), lambda i: (i, 0)),
            pl.BlockSpec((CPAD, HIDDEN_DIM), lambda i: (0, 0)),
            pl.BlockSpec((CPAD, HIDDEN_DIM), lambda i: (0, 0)),
            pl.BlockSpec((1, HIDDEN_DIM), lambda i: (0, 0)),
            pl.BlockSpec((1, HIDDEN_DIM), lambda i: (0, 0)),
        ],
        out_specs=pl.BlockSpec(memory_space=pltpu.MemorySpace.VMEM),
        out_shape=jax.ShapeDtypeStruct((BATCH, HIDDEN_DIM), jnp.float32),
        compiler_params=pltpu.CompilerParams(
            dimension_semantics=("arbitrary",),
        ),
    )(species2, w_hi, w_lo, gamma.reshape(1, HIDDEN_DIM), beta.reshape(1, HIDDEN_DIM))
    return jax.lax.broadcast_in_dim(emb, (BATCH, SEQ_LEN, HIDDEN_DIM), (0, 2))


# speedup vs baseline: 1.1904x; 1.1904x over previous
import jax
import jax.numpy as jnp
from jax.experimental import pallas as pl
from jax.experimental.pallas import tpu as pltpu

NUM_CLASSES = 1000
HIDDEN_DIM = 768
SEQ_LEN = 77
BATCH = 1024


def _body(sp_ref, w_ref, o_ref):
    w = w_ref[...]
    w_hi = w.astype(jnp.bfloat16)
    w_lo = ((w - w_hi.astype(jnp.float32)) * 512.0).astype(jnp.bfloat16)
    sp = sp_ref[...]
    cols = jax.lax.broadcasted_iota(jnp.int32, (BATCH, NUM_CLASSES), 1)
    oh = jnp.where(cols == sp, 1.0, 0.0).astype(jnp.bfloat16)
    rows = jnp.dot(
        oh, w_hi, preferred_element_type=jnp.float32
    ) + jnp.dot(oh, w_lo, preferred_element_type=jnp.float32) * (1.0 / 512.0)
    mu = jnp.mean(rows, axis=-1, keepdims=True)
    var = jnp.mean(jnp.square(rows - mu), axis=-1, keepdims=True)
    o_ref[...] = (rows - mu) * jax.lax.rsqrt(var + 1e-5)


def kernel(species, W, gamma, beta):
    species2 = species.astype(jnp.int32).reshape(BATCH, 1)
    emb = pl.pallas_call(
        _body,
        in_specs=[
            pl.BlockSpec((BATCH, 1), lambda: (0, 0)),
            pl.BlockSpec((NUM_CLASSES, HIDDEN_DIM), lambda: (0, 0)),
        ],
        out_specs=pl.BlockSpec((BATCH, HIDDEN_DIM), lambda: (0, 0)),
        out_shape=jax.ShapeDtypeStruct((BATCH, HIDDEN_DIM), jnp.float32),
    )(species2, W)
    emb = emb * gamma + beta
    return jax.lax.broadcast_in_dim(emb, (BATCH, SEQ_LEN, HIDDEN_DIM), (0, 2))


# speedup vs baseline: 1.2118x; 1.0180x over previous
import jax
import jax.numpy as jnp
from jax.experimental import pallas as pl
from jax.experimental.pallas import tpu as pltpu

NUM_CLASSES = 1000
HIDDEN_DIM = 768
SEQ_LEN = 77
BATCH = 1024


def _body(sp_ref, w_ref, o_ref):
    w = w_ref[...]
    w_hi = w.astype(jnp.bfloat16)
    w_lo = ((w - w_hi.astype(jnp.float32)) * 512.0).astype(jnp.bfloat16)
    sp = sp_ref[...][None, :]
    cls = jax.lax.broadcasted_iota(jnp.int32, (NUM_CLASSES, BATCH), 0)
    oht = jnp.where(cls == sp, 1.0, 0.0).astype(jnp.bfloat16)
    dims = (((0,), (0,)), ((), ()))
    rows = jax.lax.dot_general(
        oht, w_hi, dims, preferred_element_type=jnp.float32
    ) + jax.lax.dot_general(
        oht, w_lo, dims, preferred_element_type=jnp.float32
    ) * (1.0 / 512.0)
    mu = jnp.mean(rows, axis=-1, keepdims=True)
    var = jnp.mean(jnp.square(rows - mu), axis=-1, keepdims=True)
    o_ref[...] = (rows - mu) * jax.lax.rsqrt(var + 1e-5)


def kernel(species, W, gamma, beta):
    emb = pl.pallas_call(
        _body,
        in_specs=[
            pl.BlockSpec((BATCH,), lambda: (0,)),
            pl.BlockSpec((NUM_CLASSES, HIDDEN_DIM), lambda: (0, 0)),
        ],
        out_specs=pl.BlockSpec(memory_space=pltpu.MemorySpace.VMEM),
        out_shape=jax.ShapeDtypeStruct((BATCH, HIDDEN_DIM), jnp.float32),
    )(species.astype(jnp.int32), W)
    emb = emb * gamma + beta
    return jax.lax.broadcast_in_dim(emb, (BATCH, SEQ_LEN, HIDDEN_DIM), (0, 2))
